# baseline (speedup 1.0000x reference)
import jax
import jax.numpy as jnp
import numpy as np
from jax.experimental import pallas as pl
from jax.experimental.pallas import tpu as pltpu

_MAX_THRESHOLD = 0.95
_LOG2E = 1.4426950408889634
_SUB = 8
_DIMS = ((1024, 1024), (512, 1024), (256, 512), (128, 256))


def _net_kernel(tau_ref, x_ref, w0_ref, w1_ref, w2_ref, w3_ref, o_ref,
                ha_ref, hb_ref, awt_ref):
    B = x_ref.shape[0]
    x = x_ref[...]
    ha_ref[:, :512] = x
    ha_ref[:, 512:1024] = 1.0 - x

    for l, (w_ref, (OUT, IN)) in enumerate(
            zip((w0_ref, w1_ref, w2_ref, w3_ref), _DIMS)):
        tau_floor = float(np.log(IN - 1) + np.log(_MAX_THRESHOLD)
                          - np.log(1.0 - _MAX_THRESHOLD))
        ta = tau_ref[l, 0]
        tau = tau_floor + jnp.where(ta >= 0, ta, 0.05 * ta)
        ts = tau * _LOG2E
        inv_ts = 1.0 / ts
        w = w_ref[...]
        aw = jnp.where(w < 0.0, 0.1 * w,
                       jnp.where(w > 1.0, 1.0 + 0.1 * (w - 1.0), w))
        awt_ref[:OUT, :IN] = aw * ts
        awt_v = awt_ref.at[:OUT, :IN]
        xx = ha_ref[:, :IN]
        mh = jnp.max(jnp.abs(xx), axis=-1, keepdims=True)

        def body(j, carry, awt_v=awt_v, xx=xx, mh=mh, inv_ts=inv_ts):
            awj = awt_v[pl.ds(j * _SUB, _SUB), :]
            maj = jnp.max(jnp.abs(awj), axis=-1, keepdims=True)
            c = maj[:, :, None] * mh[None, :, :]
            t = awj[:, None, :] * xx[None, :, :]
            u = t - c
            p = jnp.exp2(u)
            den = jnp.sum(p, axis=-1, keepdims=True)
            num = jnp.sum(p * u, axis=-1, keepdims=True)
            hb_ref[pl.ds(j * _SUB, _SUB), :] = (
                1.0 - inv_ts * (c + num / den)[:, :, 0])
            return carry

        jax.lax.fori_loop(0, OUT // _SUB, body, 0)
        if l < 3:
            ha_ref[:, :OUT] = hb_ref[:OUT, :].T
        else:
            o_ref[...] = hb_ref[:OUT, :].T


def kernel(x, W0, W1, W2, W3, tau0, tau1, tau2, tau3):
    taus = jnp.stack([tau0, tau1, tau2, tau3]).reshape(4, 1)
    B = x.shape[0]
    return pl.pallas_call(
        _net_kernel,
        out_shape=jax.ShapeDtypeStruct((B, 128), jnp.float32),
        in_specs=[
            pl.BlockSpec(memory_space=pltpu.SMEM),
            pl.BlockSpec(memory_space=pltpu.VMEM),
            pl.BlockSpec(memory_space=pltpu.VMEM),
            pl.BlockSpec(memory_space=pltpu.VMEM),
            pl.BlockSpec(memory_space=pltpu.VMEM),
            pl.BlockSpec(memory_space=pltpu.VMEM),
        ],
        out_specs=pl.BlockSpec(memory_space=pltpu.VMEM),
        scratch_shapes=[
            pltpu.VMEM((B, 1024), jnp.float32),
            pltpu.VMEM((1024, B), jnp.float32),
            pltpu.VMEM((1024, 1024), jnp.float32),
        ],
        compiler_params=pltpu.CompilerParams(
            vmem_limit_bytes=52 * 1024 * 1024,
        ),
        name="logic_gate_net",
    )(taus, x, W0, W1, W2, W3)


# speedup vs baseline: 1.0667x; 1.0667x over previous
import jax
import jax.numpy as jnp
import numpy as np
from jax.experimental import pallas as pl
from jax.experimental.pallas import tpu as pltpu

_MAX_THRESHOLD = 0.95
_LOG2E = 1.4426950408889634
_SUB = 8
_UNROLL = 4
_DIMS = ((1024, 1024), (512, 1024), (256, 512), (128, 256))


def _net_kernel(tau_ref, x_ref, w0_ref, w1_ref, w2_ref, w3_ref, o_ref,
                ha_ref, hb_ref, awt_ref):
    B = x_ref.shape[0]
    x = x_ref[...]
    ha_ref[:, :512] = x
    ha_ref[:, 512:1024] = 1.0 - x

    for l, (w_ref, (OUT, IN)) in enumerate(
            zip((w0_ref, w1_ref, w2_ref, w3_ref), _DIMS)):
        tau_floor = float(np.log(IN - 1) + np.log(_MAX_THRESHOLD)
                          - np.log(1.0 - _MAX_THRESHOLD))
        ta = tau_ref[l, 0]
        tau = tau_floor + jnp.where(ta >= 0, ta, 0.05 * ta)
        ts = tau * _LOG2E
        inv_ts = 1.0 / ts
        w = w_ref[...]
        aw = jnp.where(w < 0.0, 0.1 * w,
                       jnp.where(w > 1.0, 1.0 + 0.1 * (w - 1.0), w))
        awt_ref[:OUT, :IN] = aw * ts
        awt_v = awt_ref.at[:OUT, :IN]
        xx = ha_ref[:, :IN]
        mh = jnp.max(jnp.abs(xx), axis=-1, keepdims=True)

        def body(j, carry, awt_v=awt_v, xx=xx, mh=mh, inv_ts=inv_ts):
            for k in range(_UNROLL):
                base = (j * _UNROLL + k) * _SUB
                awj = awt_v[pl.ds(base, _SUB), :]
                maj = jnp.max(jnp.abs(awj), axis=-1, keepdims=True)
                c = maj[:, :, None] * mh[None, :, :]
                t = awj[:, None, :] * xx[None, :, :]
                u = t - c
                p = jnp.exp2(u)
                den = jnp.sum(p, axis=-1, keepdims=True)
                num = jnp.sum(p * u, axis=-1, keepdims=True)
                hb_ref[pl.ds(base, _SUB), :] = (
                    1.0 - inv_ts * (c + num / den)[:, :, 0])
            return carry

        jax.lax.fori_loop(0, OUT // (_SUB * _UNROLL), body, 0)
        if l < 3:
            ha_ref[:, :OUT] = hb_ref[:OUT, :].T
        else:
            o_ref[...] = hb_ref[:OUT, :].T


def kernel(x, W0, W1, W2, W3, tau0, tau1, tau2, tau3):
    taus = jnp.stack([tau0, tau1, tau2, tau3]).reshape(4, 1)
    B = x.shape[0]
    return pl.pallas_call(
        _net_kernel,
        out_shape=jax.ShapeDtypeStruct((B, 128), jnp.float32),
        in_specs=[
            pl.BlockSpec(memory_space=pltpu.SMEM),
            pl.BlockSpec(memory_space=pltpu.VMEM),
            pl.BlockSpec(memory_space=pltpu.VMEM),
            pl.BlockSpec(memory_space=pltpu.VMEM),
            pl.BlockSpec(memory_space=pltpu.VMEM),
            pl.BlockSpec(memory_space=pltpu.VMEM),
        ],
        out_specs=pl.BlockSpec(memory_space=pltpu.VMEM),
        scratch_shapes=[
            pltpu.VMEM((B, 1024), jnp.float32),
            pltpu.VMEM((1024, B), jnp.float32),
            pltpu.VMEM((1024, 1024), jnp.float32),
        ],
        compiler_params=pltpu.CompilerParams(
            vmem_limit_bytes=52 * 1024 * 1024,
        ),
        name="logic_gate_net",
    )(taus, x, W0, W1, W2, W3)


# speedup vs baseline: 1.1358x; 1.0647x over previous
import functools

import jax
import jax.numpy as jnp
import numpy as np
from jax.experimental import pallas as pl
from jax.experimental.pallas import tpu as pltpu

_MAX_THRESHOLD = 0.95
_LOG2E = 1.4426950408889634
_SUB = 8


def _or_layer_kernel(tau_ref, x_ref, w_ref, o_ref, *, tau_floor, o_blk):
    ta = tau_ref[0, 0]
    tau = tau_floor + jnp.where(ta >= 0, ta, 0.05 * ta)
    ts = tau * _LOG2E
    inv_ts = 1.0 / ts
    x = x_ref[...]
    w = w_ref[...]
    aw = jnp.where(w < 0.0, 0.1 * w,
                   jnp.where(w > 1.0, 1.0 + 0.1 * (w - 1.0), w))
    awt = aw * ts
    C = jnp.max(jnp.abs(awt)) * jnp.max(jnp.abs(x))
    for j in range(o_blk // _SUB):
        awj = awt[j * _SUB:(j + 1) * _SUB, :]
        t = awj[:, None, :] * x[None, :, :]
        u = t - C
        p = jnp.exp2(u)
        den = jnp.sum(p, axis=-1, keepdims=True)
        num = jnp.sum(p * u, axis=-1, keepdims=True)
        o_ref[j * _SUB:(j + 1) * _SUB, :] = (
            1.0 - inv_ts * (C + (num / den)[:, :, 0]))


def _or_layer_t(h, W, tau_adder, o_blk):
    B, IN = h.shape
    OUT = W.shape[0]
    tau_floor = float(np.log(IN - 1) + np.log(_MAX_THRESHOLD)
                      - np.log(1.0 - _MAX_THRESHOLD))
    tau2d = tau_adder.reshape(1, 1)
    return pl.pallas_call(
        functools.partial(_or_layer_kernel, tau_floor=tau_floor, o_blk=o_blk),
        out_shape=jax.ShapeDtypeStruct((OUT, B), jnp.float32),
        grid=(OUT // o_blk,),
        in_specs=[
            pl.BlockSpec(memory_space=pltpu.SMEM),
            pl.BlockSpec((B, IN), lambda o: (0, 0)),
            pl.BlockSpec((o_blk, IN), lambda o: (o, 0)),
        ],
        out_specs=pl.BlockSpec((o_blk, B), lambda o: (o, 0)),
        compiler_params=pltpu.CompilerParams(
            dimension_semantics=("arbitrary",),
            vmem_limit_bytes=48 * 1024 * 1024,
        ),
        name="or_gate_layer",
    )(tau2d, h, W)


def kernel(x, W0, W1, W2, W3, tau0, tau1, tau2, tau3):
    h = jnp.concatenate([x, 1.0 - x], axis=-1)
    for W, t, blk in ((W0, tau0, 64), (W1, tau1, 64),
                      (W2, tau2, 32), (W3, tau3, 32)):
        h = _or_layer_t(h, W, t, blk).T
    return h
